# baseline (speedup 1.0000x reference)
import jax
import jax.numpy as jnp
from jax import lax
from jax.experimental import pallas as pl
from jax.experimental.pallas import tpu as pltpu

B = 1024
N = 1024
TOPO_DIM = 16
BIO_DIM = 112
D = TOPO_DIM + BIO_DIM
K = 32
HIDDEN = 8

BT = 128
BG = 16


def _topk_body(topo_ref, wt_ref, idxt_ref):
    w = wt_ref[...]
    rows = []
    for b in range(BT):
        rows.append(jnp.dot(w, topo_ref[b],
                            preferred_element_type=jnp.float32))
    sv = jnp.concatenate(rows, axis=0)
    iota_f = lax.broadcasted_iota(jnp.int32, (BT, N), 1).astype(jnp.float32)
    big = jnp.float32(N)
    cols = []
    for _ in range(K):
        m = jnp.max(sv, axis=1, keepdims=True)
        am = jnp.min(jnp.where(sv == m, iota_f, big), axis=1, keepdims=True)
        cols.append(am)
        sv = jnp.where(iota_f == am, -jnp.inf, sv)
    idx = jnp.concatenate(cols, axis=1).astype(jnp.int32)
    for j in range(BT // BG):
        idxt_ref[j] = jnp.transpose(idx[j * BG:(j + 1) * BG, :])


_topk_call = pl.pallas_call(
    _topk_body,
    grid=(B // BT,),
    in_specs=[
        pl.BlockSpec((BT, TOPO_DIM, N), lambda i: (i, 0, 0)),
        pl.BlockSpec((1, TOPO_DIM), lambda i: (0, 0)),
    ],
    out_specs=pl.BlockSpec((BT // BG, K, BG), lambda i: (i, 0, 0)),
    out_shape=jax.ShapeDtypeStruct((B // BG, K, BG), jnp.int32),
)


def _gather_body(topo_ref, bio_ref, idxt_ref, x_ref):
    iota_k = lax.broadcasted_iota(jnp.int32, (K, N), 1)
    idxt = idxt_ref[0]
    parts = []
    for b in range(BG):
        mb = jnp.where(idxt[:, b:b + 1] == iota_k, 1.0, 0.0)
        feat = jnp.concatenate([topo_ref[b], bio_ref[b]], axis=0)
        parts.append(
            lax.dot_general(mb, feat, (((1,), (1,)), ((), ())),
                            preferred_element_type=jnp.float32))
    x_ref[...] = jnp.concatenate(parts, axis=0)


_gather_call = pl.pallas_call(
    _gather_body,
    grid=(B // BG,),
    in_specs=[
        pl.BlockSpec((BG, TOPO_DIM, N), lambda i: (i, 0, 0)),
        pl.BlockSpec((BG, BIO_DIM, N), lambda i: (i, 0, 0)),
        pl.BlockSpec((1, K, BG), lambda i: (i, 0, 0)),
    ],
    out_specs=pl.BlockSpec((BG * K, D), lambda i: (i, 0)),
    out_shape=jax.ShapeDtypeStruct((B * K, D), jnp.float32),
    compiler_params=pltpu.CompilerParams(vmem_limit_bytes=60 * 1024 * 1024),
)


def _mlp_body(x_ref, w1_ref, b1_ref, w2_ref, b2_ref, y_ref,
              score_ref, loss_ref):
    h = (jnp.dot(x_ref[...], w1_ref[...], preferred_element_type=jnp.float32)
         + b1_ref[...])
    h = jnp.maximum(h, 0.0)
    s = jnp.dot(h, w2_ref[...], preferred_element_type=jnp.float32) + b2_ref[...]
    score_ref[...] = s
    yf = y_ref[...]
    npos = jnp.sum(yf)
    pw = (jnp.float32(B) - npos) / npos
    ez = jnp.exp(-jnp.abs(s))
    log1pez = jnp.log(1.0 + ez)
    ls_pos = jnp.minimum(s, 0.0) - log1pez
    ls_neg = jnp.minimum(-s, 0.0) - log1pez
    l = -(pw * yf * ls_pos + (1.0 - yf) * ls_neg)
    loss_ref[...] = jnp.sum(l, axis=0, keepdims=True) * (1.0 / B)


_mlp_call = pl.pallas_call(
    _mlp_body,
    out_shape=[
        jax.ShapeDtypeStruct((B, 1), jnp.float32),
        jax.ShapeDtypeStruct((1, 1), jnp.float32),
    ],
)


def kernel(topo_feat, bio_feat, y, W_t, b_t, W1, b1, W2, b2):
    del b_t
    topo_t = jnp.transpose(topo_feat, (0, 2, 1))
    bio_t = jnp.transpose(bio_feat, (0, 2, 1))
    idxt = _topk_call(topo_t, W_t.reshape(1, TOPO_DIM))
    x = _gather_call(topo_t, bio_t, idxt)
    x2 = x.reshape(B, K * D)
    score2, loss2 = _mlp_call(x2, W1, b1.reshape(1, HIDDEN), W2,
                              b2.reshape(1, 1),
                              y.astype(jnp.float32).reshape(B, 1))
    return (loss2.reshape(()), score2.reshape(B))


# speedup vs baseline: 1.0156x; 1.0156x over previous
import jax
import jax.numpy as jnp
from jax import lax
from jax.experimental import pallas as pl
from jax.experimental.pallas import tpu as pltpu

B = 1024
N = 1024
TOPO_DIM = 16
BIO_DIM = 112
D = TOPO_DIM + BIO_DIM
K = 32
HIDDEN = 8

BT = 128
BG = 32


def _topk_body(topo_ref, wt_ref, idxt_ref):
    w = wt_ref[...]
    rows = []
    for b in range(BT):
        rows.append(jnp.dot(w, topo_ref[b],
                            preferred_element_type=jnp.float32))
    sv = jnp.concatenate(rows, axis=0)
    iota_f = lax.broadcasted_iota(jnp.int32, (BT, N), 1).astype(jnp.float32)
    big = jnp.float32(N)
    cols = []
    for _ in range(K):
        m = jnp.max(sv, axis=1, keepdims=True)
        am = jnp.min(jnp.where(sv == m, iota_f, big), axis=1, keepdims=True)
        cols.append(am)
        sv = jnp.where(iota_f == am, -jnp.inf, sv)
    idx = jnp.concatenate(cols, axis=1).astype(jnp.int32)
    for j in range(BT // BG):
        idxt_ref[j] = jnp.transpose(idx[j * BG:(j + 1) * BG, :])


_topk_call = pl.pallas_call(
    _topk_body,
    grid=(B // BT,),
    in_specs=[
        pl.BlockSpec((BT, TOPO_DIM, N), lambda i: (i, 0, 0)),
        pl.BlockSpec((1, TOPO_DIM), lambda i: (0, 0)),
    ],
    out_specs=pl.BlockSpec((BT // BG, K, BG), lambda i: (i, 0, 0)),
    out_shape=jax.ShapeDtypeStruct((B // BG, K, BG), jnp.int32),
)


def _gather_body(topo_ref, bio_ref, idxt_ref, x_ref):
    iota_k = lax.broadcasted_iota(jnp.int32, (K, N), 1)
    idxt = idxt_ref[0]
    parts = []
    for b in range(BG):
        mb = jnp.where(idxt[:, b:b + 1] == iota_k, 1.0, 0.0)
        feat = jnp.concatenate([topo_ref[b], bio_ref[b]], axis=0)
        parts.append(
            lax.dot_general(mb, feat, (((1,), (1,)), ((), ())),
                            preferred_element_type=jnp.float32))
    x_ref[...] = jnp.concatenate(parts, axis=0)


_gather_call = pl.pallas_call(
    _gather_body,
    grid=(B // BG,),
    in_specs=[
        pl.BlockSpec((BG, TOPO_DIM, N), lambda i: (i, 0, 0)),
        pl.BlockSpec((BG, BIO_DIM, N), lambda i: (i, 0, 0)),
        pl.BlockSpec((1, K, BG), lambda i: (i, 0, 0)),
    ],
    out_specs=pl.BlockSpec((BG * K, D), lambda i: (i, 0)),
    out_shape=jax.ShapeDtypeStruct((B * K, D), jnp.float32),
    compiler_params=pltpu.CompilerParams(vmem_limit_bytes=60 * 1024 * 1024),
)


def _mlp_body(x_ref, w1_ref, b1_ref, w2_ref, b2_ref, y_ref,
              score_ref, loss_ref):
    h = (jnp.dot(x_ref[...], w1_ref[...], preferred_element_type=jnp.float32)
         + b1_ref[...])
    h = jnp.maximum(h, 0.0)
    s = jnp.dot(h, w2_ref[...], preferred_element_type=jnp.float32) + b2_ref[...]
    score_ref[...] = s
    yf = y_ref[...]
    npos = jnp.sum(yf)
    pw = (jnp.float32(B) - npos) / npos
    ez = jnp.exp(-jnp.abs(s))
    log1pez = jnp.log(1.0 + ez)
    ls_pos = jnp.minimum(s, 0.0) - log1pez
    ls_neg = jnp.minimum(-s, 0.0) - log1pez
    l = -(pw * yf * ls_pos + (1.0 - yf) * ls_neg)
    loss_ref[...] = jnp.sum(l, axis=0, keepdims=True) * (1.0 / B)


_mlp_call = pl.pallas_call(
    _mlp_body,
    out_shape=[
        jax.ShapeDtypeStruct((B, 1), jnp.float32),
        jax.ShapeDtypeStruct((1, 1), jnp.float32),
    ],
)


def kernel(topo_feat, bio_feat, y, W_t, b_t, W1, b1, W2, b2):
    del b_t
    topo_t = jnp.transpose(topo_feat, (0, 2, 1))
    bio_t = jnp.transpose(bio_feat, (0, 2, 1))
    idxt = _topk_call(topo_t, W_t.reshape(1, TOPO_DIM))
    x = _gather_call(topo_t, bio_t, idxt)
    x2 = x.reshape(B, K * D)
    score2, loss2 = _mlp_call(x2, W1, b1.reshape(1, HIDDEN), W2,
                              b2.reshape(1, 1),
                              y.astype(jnp.float32).reshape(B, 1))
    return (loss2.reshape(()), score2.reshape(B))
